# baseline (device time: 175345 ns/iter reference)
import jax
import jax.numpy as jnp
from jax import lax
from jax.experimental import pallas as pl
from jax.experimental.pallas import tpu as pltpu

N_DEV = 8
M = 1536
D = 1536
H = 3072
BH = 384
R = M // N_DEV


NK = H // BH


def _mlp_body(x_ref, wg_ref, wu_ref, wd_ref, out_ref, g_ref, u_ref):
    k = pl.program_id(0)
    x16 = x_ref[...]

    @pl.when(k < NK)
    def _():
        slot = lax.rem(k, 2)
        g_ref[slot, :, :] = jnp.dot(x16, wg_ref[...].astype(jnp.bfloat16),
                                    preferred_element_type=jnp.float32)
        u_ref[slot, :, :] = jnp.dot(x16, wu_ref[...].astype(jnp.bfloat16),
                                    preferred_element_type=jnp.float32)

    @pl.when(k > 0)
    def _():
        pslot = lax.rem(k + 1, 2)
        g = g_ref[pslot, :, :]
        u = u_ref[pslot, :, :]
        a = (g * (u * jax.nn.sigmoid(u))).astype(jnp.bfloat16)
        part = jnp.dot(a, wd_ref[...].astype(jnp.bfloat16),
                       preferred_element_type=jnp.float32)

        @pl.when(k == 1)
        def _():
            out_ref[...] = part

        @pl.when(k > 1)
        def _():
            out_ref[...] += part


def _mlp(x, wg, wu, wd):
    x16 = x.astype(jnp.bfloat16)
    return pl.pallas_call(
        _mlp_body,
        grid=(NK + 1,),
        in_specs=[
            pl.BlockSpec((M, D), lambda k: (0, 0)),
            pl.BlockSpec((D, BH), lambda k: (0, jnp.minimum(k, NK - 1))),
            pl.BlockSpec((D, BH), lambda k: (0, jnp.minimum(k, NK - 1))),
            pl.BlockSpec((BH, D), lambda k: (jnp.maximum(k - 1, 0), 0)),
        ],
        out_specs=pl.BlockSpec((M, D), lambda k: (0, 0)),
        out_shape=jax.ShapeDtypeStruct((M, D), jnp.float32),
        scratch_shapes=[
            pltpu.VMEM((2, M, BH), jnp.float32),
            pltpu.VMEM((2, M, BH), jnp.float32),
        ],
        compiler_params=pltpu.CompilerParams(
            vmem_limit_bytes=60 * 1024 * 1024),
    )(x16, wg, wu, wd)


COLS = D // 2


def _ar_body(p_ref, out_ref,
             rs_buf_a, rs_buf_b, send_buf_a, send_buf_b,
             ag_buf_a, ag_buf_b, init_a, init_b,
             rs_send_a, rs_recv_a, rs_send_b, rs_recv_b,
             ag_send_a, ag_recv_a, ag_send_b, ag_recv_b):
    i = lax.axis_index("i")
    left = lax.rem(i - 1 + N_DEV, N_DEV)
    right = lax.rem(i + 1, N_DEV)

    barrier = pltpu.get_barrier_semaphore()
    for nbr in (left, right):
        pl.semaphore_signal(barrier, inc=1, device_id=(nbr,),
                            device_id_type=pl.DeviceIdType.MESH)
    pl.semaphore_wait(barrier, 2)

    row0 = i * R
    init_a[...] = p_ref[pl.ds(row0, R), pl.ds(0, COLS)].astype(jnp.bfloat16)
    init_b[...] = p_ref[pl.ds(row0, R), pl.ds(COLS, COLS)].astype(jnp.bfloat16)

    for s in range(N_DEV - 1):
        rd_a = pltpu.make_async_remote_copy(
            src_ref=init_a if s == 0 else send_buf_a.at[s - 1],
            dst_ref=rs_buf_a.at[s],
            send_sem=rs_send_a.at[s],
            recv_sem=rs_recv_a.at[s],
            device_id=(right,),
            device_id_type=pl.DeviceIdType.MESH,
        )
        rd_b = pltpu.make_async_remote_copy(
            src_ref=init_b if s == 0 else send_buf_b.at[s - 1],
            dst_ref=rs_buf_b.at[s],
            send_sem=rs_send_b.at[s],
            recv_sem=rs_recv_b.at[s],
            device_id=(left,),
            device_id_type=pl.DeviceIdType.MESH,
        )
        rd_a.start()
        rd_b.start()
        rd_a.wait()
        rd_b.wait()
        ca = lax.rem(i - s - 1 + N_DEV, N_DEV)
        cb = lax.rem(i + s + 1, N_DEV)
        acc_a = (rs_buf_a[s, :, :].astype(jnp.float32)
                 + p_ref[pl.ds(ca * R, R), pl.ds(0, COLS)])
        acc_b = (rs_buf_b[s, :, :].astype(jnp.float32)
                 + p_ref[pl.ds(cb * R, R), pl.ds(COLS, COLS)])
        send_buf_a[s, :, :] = acc_a.astype(jnp.bfloat16)
        send_buf_b[s, :, :] = acc_b.astype(jnp.bfloat16)
        if s == N_DEV - 2:
            out_ref[pl.ds(ca * R, R), pl.ds(0, COLS)] = acc_a
            out_ref[pl.ds(cb * R, R), pl.ds(COLS, COLS)] = acc_b

    def _ag_start(t):
        rd_a = pltpu.make_async_remote_copy(
            src_ref=send_buf_a.at[N_DEV - 2] if t == 0 else ag_buf_a.at[t - 1],
            dst_ref=ag_buf_a.at[t],
            send_sem=ag_send_a.at[t],
            recv_sem=ag_recv_a.at[t],
            device_id=(right,),
            device_id_type=pl.DeviceIdType.MESH,
        )
        rd_b = pltpu.make_async_remote_copy(
            src_ref=send_buf_b.at[N_DEV - 2] if t == 0 else ag_buf_b.at[t - 1],
            dst_ref=ag_buf_b.at[t],
            send_sem=ag_send_b.at[t],
            recv_sem=ag_recv_b.at[t],
            device_id=(left,),
            device_id_type=pl.DeviceIdType.MESH,
        )
        rd_a.start()
        rd_b.start()
        return rd_a, rd_b

    pending = _ag_start(0)
    for t in range(N_DEV - 1):
        rd_a, rd_b = pending
        rd_a.wait()
        rd_b.wait()
        if t < N_DEV - 2:
            pending = _ag_start(t + 1)
        ca = lax.rem(i - t + N_DEV, N_DEV)
        cb = lax.rem(i + t, N_DEV)
        out_ref[pl.ds(ca * R, R), pl.ds(0, COLS)] = (
            ag_buf_a[t, :, :].astype(jnp.float32))
        out_ref[pl.ds(cb * R, R), pl.ds(COLS, COLS)] = (
            ag_buf_b[t, :, :].astype(jnp.float32))


def _all_reduce(partial):
    n_steps = N_DEV - 1
    buf = pltpu.VMEM((n_steps, R, COLS), jnp.bfloat16)
    sems = pltpu.SemaphoreType.DMA((n_steps,))
    return pl.pallas_call(
        _ar_body,
        out_shape=jax.ShapeDtypeStruct((M, D), jnp.float32),
        in_specs=[pl.BlockSpec(memory_space=pltpu.VMEM)],
        out_specs=pl.BlockSpec(memory_space=pltpu.VMEM),
        scratch_shapes=(
            [buf] * 6
            + [pltpu.VMEM((R, COLS), jnp.bfloat16)] * 2
            + [sems] * 8
        ),
        compiler_params=pltpu.CompilerParams(collective_id=0),
    )(partial)


def kernel(x, Wg, Wu, Wd):
    partial = _mlp(x, Wg, Wu, Wd)
    return _all_reduce(partial)


# device time: 155762 ns/iter; 1.1257x vs baseline; 1.1257x over previous
import jax
import jax.numpy as jnp
from jax import lax
from jax.experimental import pallas as pl
from jax.experimental.pallas import tpu as pltpu

N_DEV = 8
M = 1536
D = 1536
H = 3072
BH = 384
R = M // N_DEV


def _mlp_body(x_ref, wg_ref, wu_ref, wd_ref, out_ref, x16_ref):
    k = pl.program_id(0)

    @pl.when(k == 0)
    def _():
        x16_ref[...] = x_ref[...].astype(jnp.bfloat16)

    x16 = x16_ref[...]
    g = jnp.dot(x16, wg_ref[...].astype(jnp.bfloat16),
                preferred_element_type=jnp.float32)
    u = jnp.dot(x16, wu_ref[...].astype(jnp.bfloat16),
                preferred_element_type=jnp.float32)
    a = g * (u * jax.nn.sigmoid(u))
    part = jnp.dot(a.astype(jnp.bfloat16), wd_ref[...].astype(jnp.bfloat16),
                   preferred_element_type=jnp.float32)

    @pl.when(k == 0)
    def _():
        out_ref[...] = part

    @pl.when(k > 0)
    def _():
        out_ref[...] += part


def _mlp(x, wg, wu, wd):
    return pl.pallas_call(
        _mlp_body,
        grid=(H // BH,),
        in_specs=[
            pl.BlockSpec((M, D), lambda k: (0, 0)),
            pl.BlockSpec((D, BH), lambda k: (0, k)),
            pl.BlockSpec((D, BH), lambda k: (0, k)),
            pl.BlockSpec((BH, D), lambda k: (k, 0)),
        ],
        out_specs=pl.BlockSpec((M, D), lambda k: (0, 0)),
        out_shape=jax.ShapeDtypeStruct((M, D), jnp.float32),
        scratch_shapes=[pltpu.VMEM((M, D), jnp.bfloat16)],
        compiler_params=pltpu.CompilerParams(
            vmem_limit_bytes=60 * 1024 * 1024),
    )(x, wg, wu, wd)


COLS = D // 2

MX, MY, MZ = 1, 3, 4
DIMS_A = (MX, MY, MZ)
DIMS_B = (MY, MZ, MX)


def _ar_body(p_ref, out_ref,
             snd1_a, snd2_a, snd3_a, rcv1_a, rcv2_a, rcv3_a,
             acc1_a, acc2_a, g_a,
             snd1_b, snd2_b, snd3_b, rcv1_b, rcv2_b, rcv3_b,
             acc1_b, acc2_b, g_b,
             rs_send_a, rs_recv_a, rs_send_b, rs_recv_b,
             ag_send_a, ag_recv_a, ag_send_b, ag_recv_b):
    i = lax.axis_index("i")

    h_a = (i ^ (i >> 1)) & 1
    q_a = i & 1
    r_a = (i >> 2) & 1
    c_a = 4 * h_a + 2 * q_a + r_a
    h_b = (i >> 1) & 1
    q_b = (i >> 2) & 1
    r_b = i & 1
    c_b = 4 * h_b + 2 * q_b + r_b

    barrier = pltpu.get_barrier_semaphore()
    for m in (MX, MY, MZ):
        pl.semaphore_signal(barrier, inc=1, device_id=(i ^ m,),
                            device_id_type=pl.DeviceIdType.MESH)
    pl.semaphore_wait(barrier, 3)

    def xchg(src, dst, send_sem, recv_sem, mask):
        rd = pltpu.make_async_remote_copy(
            src_ref=src, dst_ref=dst, send_sem=send_sem, recv_sem=recv_sem,
            device_id=(i ^ mask,), device_id_type=pl.DeviceIdType.MESH)
        rd.start()
        return rd

    bf16 = jnp.bfloat16
    f32 = jnp.float32

    snd1_a[...] = p_ref[pl.ds((1 - h_a) * 768, 768), pl.ds(0, COLS)].astype(bf16)
    snd1_b[...] = p_ref[pl.ds((1 - h_b) * 768, 768), pl.ds(COLS, COLS)].astype(bf16)
    ra = xchg(snd1_a, rcv1_a, rs_send_a.at[0], rs_recv_a.at[0], DIMS_A[0])
    rb = xchg(snd1_b, rcv1_b, rs_send_b.at[0], rs_recv_b.at[0], DIMS_B[0])
    ra.wait()
    rb.wait()
    acc1_a[...] = (rcv1_a[...].astype(f32)
                   + p_ref[pl.ds(h_a * 768, 768), pl.ds(0, COLS)])
    acc1_b[...] = (rcv1_b[...].astype(f32)
                   + p_ref[pl.ds(h_b * 768, 768), pl.ds(COLS, COLS)])

    snd2_a[...] = acc1_a[pl.ds((1 - q_a) * 384, 384), :].astype(bf16)
    snd2_b[...] = acc1_b[pl.ds((1 - q_b) * 384, 384), :].astype(bf16)
    ra = xchg(snd2_a, rcv2_a, rs_send_a.at[1], rs_recv_a.at[1], DIMS_A[1])
    rb = xchg(snd2_b, rcv2_b, rs_send_b.at[1], rs_recv_b.at[1], DIMS_B[1])
    ra.wait()
    rb.wait()
    acc2_a[...] = rcv2_a[...].astype(f32) + acc1_a[pl.ds(q_a * 384, 384), :]
    acc2_b[...] = rcv2_b[...].astype(f32) + acc1_b[pl.ds(q_b * 384, 384), :]

    snd3_a[...] = acc2_a[pl.ds((1 - r_a) * 192, 192), :].astype(bf16)
    snd3_b[...] = acc2_b[pl.ds((1 - r_b) * 192, 192), :].astype(bf16)
    ra = xchg(snd3_a, rcv3_a, rs_send_a.at[2], rs_recv_a.at[2], DIMS_A[2])
    rb = xchg(snd3_b, rcv3_b, rs_send_b.at[2], rs_recv_b.at[2], DIMS_B[2])
    ra.wait()
    rb.wait()
    acc3_a = rcv3_a[...].astype(f32) + acc2_a[pl.ds(r_a * 192, 192), :]
    acc3_b = rcv3_b[...].astype(f32) + acc2_b[pl.ds(r_b * 192, 192), :]
    out_ref[pl.ds(c_a * 192, 192), pl.ds(0, COLS)] = acc3_a
    out_ref[pl.ds(c_b * 192, 192), pl.ds(COLS, COLS)] = acc3_b
    g_a[pl.ds(c_a * 192, 192), :] = acc3_a.astype(bf16)
    g_b[pl.ds(c_b * 192, 192), :] = acc3_b.astype(bf16)

    ra = xchg(g_a.at[pl.ds(c_a * 192, 192), :], g_a.at[pl.ds(c_a * 192, 192), :],
              ag_send_a.at[0], ag_recv_a.at[0], DIMS_A[2])
    rb = xchg(g_b.at[pl.ds(c_b * 192, 192), :], g_b.at[pl.ds(c_b * 192, 192), :],
              ag_send_b.at[0], ag_recv_b.at[0], DIMS_B[2])
    ra.wait()
    rb.wait()

    s2_a = (4 * h_a + 2 * q_a) * 192
    s2_b = (4 * h_b + 2 * q_b) * 192
    ra = xchg(g_a.at[pl.ds(s2_a, 384), :], g_a.at[pl.ds(s2_a, 384), :],
              ag_send_a.at[1], ag_recv_a.at[1], DIMS_A[1])
    rb = xchg(g_b.at[pl.ds(s2_b, 384), :], g_b.at[pl.ds(s2_b, 384), :],
              ag_send_b.at[1], ag_recv_b.at[1], DIMS_B[1])
    p1_a = (4 * h_a + 2 * q_a + 1 - r_a) * 192
    p1_b = (4 * h_b + 2 * q_b + 1 - r_b) * 192
    out_ref[pl.ds(p1_a, 192), pl.ds(0, COLS)] = (
        g_a[pl.ds(p1_a, 192), :].astype(f32))
    out_ref[pl.ds(p1_b, 192), pl.ds(COLS, COLS)] = (
        g_b[pl.ds(p1_b, 192), :].astype(f32))
    ra.wait()
    rb.wait()

    ra = xchg(g_a.at[pl.ds(h_a * 768, 768), :], g_a.at[pl.ds(h_a * 768, 768), :],
              ag_send_a.at[2], ag_recv_a.at[2], DIMS_A[0])
    rb = xchg(g_b.at[pl.ds(h_b * 768, 768), :], g_b.at[pl.ds(h_b * 768, 768), :],
              ag_send_b.at[2], ag_recv_b.at[2], DIMS_B[0])
    p2_a = (4 * h_a + 2 * (1 - q_a)) * 192
    p2_b = (4 * h_b + 2 * (1 - q_b)) * 192
    out_ref[pl.ds(p2_a, 384), pl.ds(0, COLS)] = (
        g_a[pl.ds(p2_a, 384), :].astype(f32))
    out_ref[pl.ds(p2_b, 384), pl.ds(COLS, COLS)] = (
        g_b[pl.ds(p2_b, 384), :].astype(f32))
    ra.wait()
    rb.wait()

    out_ref[pl.ds((1 - h_a) * 768, 768), pl.ds(0, COLS)] = (
        g_a[pl.ds((1 - h_a) * 768, 768), :].astype(f32))
    out_ref[pl.ds((1 - h_b) * 768, 768), pl.ds(COLS, COLS)] = (
        g_b[pl.ds((1 - h_b) * 768, 768), :].astype(f32))


def _all_reduce(partial):
    half_bufs = [
        pltpu.VMEM((768, COLS), jnp.bfloat16),
        pltpu.VMEM((384, COLS), jnp.bfloat16),
        pltpu.VMEM((192, COLS), jnp.bfloat16),
        pltpu.VMEM((768, COLS), jnp.bfloat16),
        pltpu.VMEM((384, COLS), jnp.bfloat16),
        pltpu.VMEM((192, COLS), jnp.bfloat16),
        pltpu.VMEM((768, COLS), jnp.float32),
        pltpu.VMEM((384, COLS), jnp.float32),
        pltpu.VMEM((M, COLS), jnp.bfloat16),
    ]
    sems = pltpu.SemaphoreType.DMA((3,))
    return pl.pallas_call(
        _ar_body,
        out_shape=jax.ShapeDtypeStruct((M, D), jnp.float32),
        in_specs=[pl.BlockSpec(memory_space=pltpu.VMEM)],
        out_specs=pl.BlockSpec(memory_space=pltpu.VMEM),
        scratch_shapes=half_bufs + half_bufs + [sems] * 8,
        compiler_params=pltpu.CompilerParams(
            collective_id=0, vmem_limit_bytes=60 * 1024 * 1024),
    )(partial)


def kernel(x, Wg, Wu, Wd):
    partial = _mlp(x, Wg, Wu, Wd)
    return _all_reduce(partial)


# device time: 141156 ns/iter; 1.2422x vs baseline; 1.1035x over previous
import jax
import jax.numpy as jnp
from jax import lax
from jax.experimental import pallas as pl
from jax.experimental.pallas import tpu as pltpu

N_DEV = 8
M = 1536
D = 1536
H = 3072
BH = 512
R = M // N_DEV


def _mlp_body(x_ref, wg_ref, wu_ref, wd_ref, out_ref, x16_ref):
    k = pl.program_id(0)

    @pl.when(k == 0)
    def _():
        x16_ref[...] = x_ref[...].astype(jnp.bfloat16)

    x16 = x16_ref[...]
    g = jnp.dot(x16, wg_ref[...].astype(jnp.bfloat16),
                preferred_element_type=jnp.float32)
    u = jnp.dot(x16, wu_ref[...].astype(jnp.bfloat16),
                preferred_element_type=jnp.float32)
    a = g * (u * jax.nn.sigmoid(u))
    part = jnp.dot(a.astype(jnp.bfloat16), wd_ref[...].astype(jnp.bfloat16),
                   preferred_element_type=jnp.float32)

    @pl.when(k == 0)
    def _():
        out_ref[...] = part

    @pl.when(k > 0)
    def _():
        out_ref[...] += part


def _mlp(x, wg, wu, wd):
    return pl.pallas_call(
        _mlp_body,
        grid=(H // BH,),
        in_specs=[
            pl.BlockSpec(memory_space=pltpu.VMEM),
            pl.BlockSpec((D, BH), lambda k: (0, k)),
            pl.BlockSpec((D, BH), lambda k: (0, k)),
            pl.BlockSpec((BH, D), lambda k: (k, 0)),
        ],
        out_specs=pl.BlockSpec(memory_space=pltpu.VMEM),
        out_shape=jax.ShapeDtypeStruct((M, D), jnp.float32),
        scratch_shapes=[pltpu.VMEM((M, D), jnp.bfloat16)],
        compiler_params=pltpu.CompilerParams(
            vmem_limit_bytes=60 * 1024 * 1024),
    )(x, wg, wu, wd)


COLS = D // 2

MX, MY, MZ = 1, 3, 4
DIMS_A = (MX, MY, MZ)
DIMS_B = (MY, MZ, MX)


def _ar_body(p_ref, out_ref,
             snd1_a, snd2_a, snd3_a, rcv1_a, rcv2_a, rcv3_a,
             acc1_a, acc2_a, g_a,
             snd1_b, snd2_b, snd3_b, rcv1_b, rcv2_b, rcv3_b,
             acc1_b, acc2_b, g_b,
             rs_send_a, rs_recv_a, rs_send_b, rs_recv_b,
             ag_send_a, ag_recv_a, ag_send_b, ag_recv_b):
    i = lax.axis_index("i")

    h_a = (i ^ (i >> 1)) & 1
    q_a = i & 1
    r_a = (i >> 2) & 1
    c_a = 4 * h_a + 2 * q_a + r_a
    h_b = (i >> 1) & 1
    q_b = (i >> 2) & 1
    r_b = i & 1
    c_b = 4 * h_b + 2 * q_b + r_b

    barrier = pltpu.get_barrier_semaphore()
    for m in (MX, MY, MZ):
        pl.semaphore_signal(barrier, inc=1, device_id=(i ^ m,),
                            device_id_type=pl.DeviceIdType.MESH)
    pl.semaphore_wait(barrier, 3)

    def xchg(src, dst, send_sem, recv_sem, mask):
        rd = pltpu.make_async_remote_copy(
            src_ref=src, dst_ref=dst, send_sem=send_sem, recv_sem=recv_sem,
            device_id=(i ^ mask,), device_id_type=pl.DeviceIdType.MESH)
        rd.start()
        return rd

    bf16 = jnp.bfloat16
    f32 = jnp.float32

    snd1_a[...] = p_ref[pl.ds((1 - h_a) * 768, 768), pl.ds(0, COLS)].astype(bf16)
    snd1_b[...] = p_ref[pl.ds((1 - h_b) * 768, 768), pl.ds(COLS, COLS)].astype(bf16)
    ra = xchg(snd1_a, rcv1_a, rs_send_a.at[0], rs_recv_a.at[0], DIMS_A[0])
    rb = xchg(snd1_b, rcv1_b, rs_send_b.at[0], rs_recv_b.at[0], DIMS_B[0])
    ra.wait()
    rb.wait()
    acc1_a[...] = (rcv1_a[...].astype(f32)
                   + p_ref[pl.ds(h_a * 768, 768), pl.ds(0, COLS)])
    acc1_b[...] = (rcv1_b[...].astype(f32)
                   + p_ref[pl.ds(h_b * 768, 768), pl.ds(COLS, COLS)])

    snd2_a[...] = acc1_a[pl.ds((1 - q_a) * 384, 384), :].astype(bf16)
    snd2_b[...] = acc1_b[pl.ds((1 - q_b) * 384, 384), :].astype(bf16)
    ra = xchg(snd2_a, rcv2_a, rs_send_a.at[1], rs_recv_a.at[1], DIMS_A[1])
    rb = xchg(snd2_b, rcv2_b, rs_send_b.at[1], rs_recv_b.at[1], DIMS_B[1])
    ra.wait()
    rb.wait()
    acc2_a[...] = rcv2_a[...].astype(f32) + acc1_a[pl.ds(q_a * 384, 384), :]
    acc2_b[...] = rcv2_b[...].astype(f32) + acc1_b[pl.ds(q_b * 384, 384), :]

    snd3_a[...] = acc2_a[pl.ds((1 - r_a) * 192, 192), :].astype(bf16)
    snd3_b[...] = acc2_b[pl.ds((1 - r_b) * 192, 192), :].astype(bf16)
    ra = xchg(snd3_a, rcv3_a, rs_send_a.at[2], rs_recv_a.at[2], DIMS_A[2])
    rb = xchg(snd3_b, rcv3_b, rs_send_b.at[2], rs_recv_b.at[2], DIMS_B[2])
    ra.wait()
    rb.wait()
    acc3_a = rcv3_a[...].astype(f32) + acc2_a[pl.ds(r_a * 192, 192), :]
    acc3_b = rcv3_b[...].astype(f32) + acc2_b[pl.ds(r_b * 192, 192), :]
    out_ref[pl.ds(c_a * 192, 192), pl.ds(0, COLS)] = acc3_a
    out_ref[pl.ds(c_b * 192, 192), pl.ds(COLS, COLS)] = acc3_b
    g_a[pl.ds(c_a * 192, 192), :] = acc3_a.astype(bf16)
    g_b[pl.ds(c_b * 192, 192), :] = acc3_b.astype(bf16)

    ra = xchg(g_a.at[pl.ds(c_a * 192, 192), :], g_a.at[pl.ds(c_a * 192, 192), :],
              ag_send_a.at[0], ag_recv_a.at[0], DIMS_A[2])
    rb = xchg(g_b.at[pl.ds(c_b * 192, 192), :], g_b.at[pl.ds(c_b * 192, 192), :],
              ag_send_b.at[0], ag_recv_b.at[0], DIMS_B[2])
    ra.wait()
    rb.wait()

    s2_a = (4 * h_a + 2 * q_a) * 192
    s2_b = (4 * h_b + 2 * q_b) * 192
    ra = xchg(g_a.at[pl.ds(s2_a, 384), :], g_a.at[pl.ds(s2_a, 384), :],
              ag_send_a.at[1], ag_recv_a.at[1], DIMS_A[1])
    rb = xchg(g_b.at[pl.ds(s2_b, 384), :], g_b.at[pl.ds(s2_b, 384), :],
              ag_send_b.at[1], ag_recv_b.at[1], DIMS_B[1])
    p1_a = (4 * h_a + 2 * q_a + 1 - r_a) * 192
    p1_b = (4 * h_b + 2 * q_b + 1 - r_b) * 192
    out_ref[pl.ds(p1_a, 192), pl.ds(0, COLS)] = (
        g_a[pl.ds(p1_a, 192), :].astype(f32))
    out_ref[pl.ds(p1_b, 192), pl.ds(COLS, COLS)] = (
        g_b[pl.ds(p1_b, 192), :].astype(f32))
    ra.wait()
    rb.wait()

    ra = xchg(g_a.at[pl.ds(h_a * 768, 768), :], g_a.at[pl.ds(h_a * 768, 768), :],
              ag_send_a.at[2], ag_recv_a.at[2], DIMS_A[0])
    rb = xchg(g_b.at[pl.ds(h_b * 768, 768), :], g_b.at[pl.ds(h_b * 768, 768), :],
              ag_send_b.at[2], ag_recv_b.at[2], DIMS_B[0])
    p2_a = (4 * h_a + 2 * (1 - q_a)) * 192
    p2_b = (4 * h_b + 2 * (1 - q_b)) * 192
    out_ref[pl.ds(p2_a, 384), pl.ds(0, COLS)] = (
        g_a[pl.ds(p2_a, 384), :].astype(f32))
    out_ref[pl.ds(p2_b, 384), pl.ds(COLS, COLS)] = (
        g_b[pl.ds(p2_b, 384), :].astype(f32))
    ra.wait()
    rb.wait()

    out_ref[pl.ds((1 - h_a) * 768, 768), pl.ds(0, COLS)] = (
        g_a[pl.ds((1 - h_a) * 768, 768), :].astype(f32))
    out_ref[pl.ds((1 - h_b) * 768, 768), pl.ds(COLS, COLS)] = (
        g_b[pl.ds((1 - h_b) * 768, 768), :].astype(f32))


def _all_reduce(partial):
    half_bufs = [
        pltpu.VMEM((768, COLS), jnp.bfloat16),
        pltpu.VMEM((384, COLS), jnp.bfloat16),
        pltpu.VMEM((192, COLS), jnp.bfloat16),
        pltpu.VMEM((768, COLS), jnp.bfloat16),
        pltpu.VMEM((384, COLS), jnp.bfloat16),
        pltpu.VMEM((192, COLS), jnp.bfloat16),
        pltpu.VMEM((768, COLS), jnp.float32),
        pltpu.VMEM((384, COLS), jnp.float32),
        pltpu.VMEM((M, COLS), jnp.bfloat16),
    ]
    sems = pltpu.SemaphoreType.DMA((3,))
    return pl.pallas_call(
        _ar_body,
        out_shape=jax.ShapeDtypeStruct((M, D), jnp.float32),
        in_specs=[pl.BlockSpec(memory_space=pltpu.VMEM)],
        out_specs=pl.BlockSpec(memory_space=pltpu.VMEM),
        scratch_shapes=half_bufs + half_bufs + [sems] * 8,
        compiler_params=pltpu.CompilerParams(
            collective_id=0, vmem_limit_bytes=60 * 1024 * 1024),
    )(partial)


def kernel(x, Wg, Wu, Wd):
    partial = _mlp(x, Wg, Wu, Wd)
    return _all_reduce(partial)


# device time: 126181 ns/iter; 1.3896x vs baseline; 1.1187x over previous
import jax
import jax.numpy as jnp
from jax import lax
from jax.experimental import pallas as pl
from jax.experimental.pallas import tpu as pltpu

N_DEV = 8
M = 1536
D = 1536
H = 3072
BH = 512
R = M // N_DEV


def _mlp_body(x_ref, wg_ref, wu_ref, wd_ref, out_ref, x16_ref):
    k = pl.program_id(0)

    @pl.when(k == 0)
    def _():
        x16_ref[...] = x_ref[...].astype(jnp.bfloat16)

    x16 = x16_ref[...]
    g = jnp.dot(x16, wg_ref[...].astype(jnp.bfloat16),
                preferred_element_type=jnp.float32)
    u = jnp.dot(x16, wu_ref[...].astype(jnp.bfloat16),
                preferred_element_type=jnp.float32)
    a = g * (u * jax.nn.sigmoid(u))
    part = jnp.dot(a.astype(jnp.bfloat16), wd_ref[...].astype(jnp.bfloat16),
                   preferred_element_type=jnp.float32)

    @pl.when(k == 0)
    def _():
        out_ref[...] = part

    @pl.when(k > 0)
    def _():
        out_ref[...] += part


def _mlp(x, wg, wu, wd):
    return pl.pallas_call(
        _mlp_body,
        grid=(H // BH,),
        in_specs=[
            pl.BlockSpec(memory_space=pltpu.VMEM),
            pl.BlockSpec((D, BH), lambda k: (0, k)),
            pl.BlockSpec((D, BH), lambda k: (0, k)),
            pl.BlockSpec((BH, D), lambda k: (k, 0)),
        ],
        out_specs=pl.BlockSpec(memory_space=pltpu.VMEM),
        out_shape=jax.ShapeDtypeStruct((M, D), jnp.float32),
        scratch_shapes=[pltpu.VMEM((M, D), jnp.bfloat16)],
        compiler_params=pltpu.CompilerParams(
            vmem_limit_bytes=60 * 1024 * 1024),
    )(x, wg, wu, wd)


NG = 3
GCOLS = D // NG
DIM_ORDERS = ((1, 3, 4), (3, 4, 1), (4, 1, 3))


def _ar_body(p_ref, out_ref, *refs):
    bufs = [refs[9 * g:9 * (g + 1)] for g in range(NG)]
    sems = [refs[9 * NG + 4 * g:9 * NG + 4 * (g + 1)] for g in range(NG)]

    i = lax.axis_index("i")
    b0 = i & 1
    b1 = (i >> 1) & 1
    b2 = (i >> 2) & 1
    par = (i ^ (i >> 1)) & 1
    roles = ((par, b0, b2), (b1, b2, b0), (b2, par, b1))

    barrier = pltpu.get_barrier_semaphore()
    for m in (1, 3, 4):
        pl.semaphore_signal(barrier, inc=1, device_id=(i ^ m,),
                            device_id_type=pl.DeviceIdType.MESH)
    pl.semaphore_wait(barrier, 3)

    def xchg(src, dst, send_sem, recv_sem, mask):
        rd = pltpu.make_async_remote_copy(
            src_ref=src, dst_ref=dst, send_sem=send_sem, recv_sem=recv_sem,
            device_id=(i ^ mask,), device_id_type=pl.DeviceIdType.MESH)
        rd.start()
        return rd

    bf16 = jnp.bfloat16
    f32 = jnp.float32
    co = [g * GCOLS for g in range(NG)]

    rds = []
    for g in range(NG):
        snd1, _, _, rcv1, _, _, _, _, _ = bufs[g]
        h = roles[g][0]
        snd1[...] = p_ref[pl.ds((1 - h) * 768, 768),
                          pl.ds(co[g], GCOLS)].astype(bf16)
        rds.append(xchg(snd1, rcv1, sems[g][0].at[0], sems[g][1].at[0],
                        DIM_ORDERS[g][0]))
    for g in range(NG):
        rds[g].wait()
        _, _, _, rcv1, _, _, acc1, _, _ = bufs[g]
        h = roles[g][0]
        acc1[...] = (rcv1[...].astype(f32)
                     + p_ref[pl.ds(h * 768, 768), pl.ds(co[g], GCOLS)])

    rds = []
    for g in range(NG):
        _, snd2, _, _, rcv2, _, acc1, _, _ = bufs[g]
        q = roles[g][1]
        snd2[...] = acc1[pl.ds((1 - q) * 384, 384), :].astype(bf16)
        rds.append(xchg(snd2, rcv2, sems[g][0].at[1], sems[g][1].at[1],
                        DIM_ORDERS[g][1]))
    for g in range(NG):
        rds[g].wait()
        _, _, _, _, rcv2, _, acc1, acc2, _ = bufs[g]
        q = roles[g][1]
        acc2[...] = rcv2[...].astype(f32) + acc1[pl.ds(q * 384, 384), :]

    rds = []
    for g in range(NG):
        _, _, snd3, _, _, _, _, acc2, _ = bufs[g]
        r = roles[g][2]
        snd3[...] = acc2[pl.ds((1 - r) * 192, 192), :].astype(bf16)
        rds.append(xchg(snd3, bufs[g][5], sems[g][0].at[2], sems[g][1].at[2],
                        DIM_ORDERS[g][2]))
    own = []
    for g in range(NG):
        rds[g].wait()
        _, _, _, _, _, rcv3, _, acc2, gbuf = bufs[g]
        h, q, r = roles[g]
        c = 4 * h + 2 * q + r
        own.append(c)
        acc3 = rcv3[...].astype(f32) + acc2[pl.ds(r * 192, 192), :]
        out_ref[pl.ds(c * 192, 192), pl.ds(co[g], GCOLS)] = acc3
        gbuf[pl.ds(c * 192, 192), :] = acc3.astype(bf16)

    rds = []
    for g in range(NG):
        gbuf = bufs[g][8]
        seg = own[g] * 192
        rds.append(xchg(gbuf.at[pl.ds(seg, 192), :],
                        gbuf.at[pl.ds(seg, 192), :],
                        sems[g][2].at[0], sems[g][3].at[0],
                        DIM_ORDERS[g][2]))
    for g in range(NG):
        rds[g].wait()
    rds = []
    for g in range(NG):
        gbuf = bufs[g][8]
        h, q, r = roles[g]
        seg = (4 * h + 2 * q) * 192
        rds.append(xchg(gbuf.at[pl.ds(seg, 384), :],
                        gbuf.at[pl.ds(seg, 384), :],
                        sems[g][2].at[1], sems[g][3].at[1],
                        DIM_ORDERS[g][1]))
    for g in range(NG):
        gbuf = bufs[g][8]
        h, q, r = roles[g]
        p1 = (4 * h + 2 * q + 1 - r) * 192
        out_ref[pl.ds(p1, 192), pl.ds(co[g], GCOLS)] = (
            gbuf[pl.ds(p1, 192), :].astype(f32))
    for g in range(NG):
        rds[g].wait()
    rds = []
    for g in range(NG):
        gbuf = bufs[g][8]
        h = roles[g][0]
        rds.append(xchg(gbuf.at[pl.ds(h * 768, 768), :],
                        gbuf.at[pl.ds(h * 768, 768), :],
                        sems[g][2].at[2], sems[g][3].at[2],
                        DIM_ORDERS[g][0]))
    for g in range(NG):
        gbuf = bufs[g][8]
        h, q, r = roles[g]
        p2 = (4 * h + 2 * (1 - q)) * 192
        out_ref[pl.ds(p2, 384), pl.ds(co[g], GCOLS)] = (
            gbuf[pl.ds(p2, 384), :].astype(f32))
    for g in range(NG):
        rds[g].wait()
    for g in range(NG):
        gbuf = bufs[g][8]
        h = roles[g][0]
        out_ref[pl.ds((1 - h) * 768, 768), pl.ds(co[g], GCOLS)] = (
            gbuf[pl.ds((1 - h) * 768, 768), :].astype(f32))


def _all_reduce(partial):
    group_bufs = [
        pltpu.VMEM((768, GCOLS), jnp.bfloat16),
        pltpu.VMEM((384, GCOLS), jnp.bfloat16),
        pltpu.VMEM((192, GCOLS), jnp.bfloat16),
        pltpu.VMEM((768, GCOLS), jnp.bfloat16),
        pltpu.VMEM((384, GCOLS), jnp.bfloat16),
        pltpu.VMEM((192, GCOLS), jnp.bfloat16),
        pltpu.VMEM((768, GCOLS), jnp.float32),
        pltpu.VMEM((384, GCOLS), jnp.float32),
        pltpu.VMEM((M, GCOLS), jnp.bfloat16),
    ]
    sems = pltpu.SemaphoreType.DMA((3,))
    return pl.pallas_call(
        _ar_body,
        out_shape=jax.ShapeDtypeStruct((M, D), jnp.float32),
        in_specs=[pl.BlockSpec(memory_space=pltpu.VMEM)],
        out_specs=pl.BlockSpec(memory_space=pltpu.VMEM),
        scratch_shapes=group_bufs * NG + [sems] * (4 * NG),
        compiler_params=pltpu.CompilerParams(
            collective_id=0, vmem_limit_bytes=60 * 1024 * 1024),
    )(partial)


def kernel(x, Wg, Wu, Wd):
    partial = _mlp(x, Wg, Wu, Wd)
    return _all_reduce(partial)
